# Initial kernel scaffold; baseline (speedup 1.0000x reference)
import jax
import jax.numpy as jnp
from jax.experimental import pallas as pl


def kernel(h, W, b, segment_ids, perm):
    raise NotImplementedError("write your pallas kernel here")


# speedup vs baseline: 2.0417x; 2.0417x over previous
import jax
import jax.numpy as jnp
from jax.experimental import pallas as pl
from jax.experimental.pallas import tpu as pltpu

_N = 100000
_D = 128
_B = 1000
_BP = 1008
_TN = 400
_GRID = _N // _TN


def _segsum_kernel(seg_ref, h_ref, sum_ref, cnt_ref):
    i = pl.program_id(0)

    @pl.when(i == 0)
    def _init():
        sum_ref[...] = jnp.zeros_like(sum_ref)
        cnt_ref[...] = jnp.zeros_like(cnt_ref)

    seg = seg_ref[0, 0, :]
    pt = (jax.lax.broadcasted_iota(jnp.int32, (_BP, _TN), 0)
          == seg[None, :]).astype(jnp.float32)
    sum_ref[...] += jnp.dot(pt, h_ref[...],
                            preferred_element_type=jnp.float32)
    oh = (jax.lax.broadcasted_iota(jnp.int32, (_TN, _BP), 1)
          == seg[:, None]).astype(jnp.float32)
    cnt_ref[...] += jnp.broadcast_to(jnp.sum(oh, axis=0, keepdims=True),
                                     (8, _BP))


def _table_kernel(sumt_ref, cnt_ref, w_mat_ref, vt_ref, w_ref):
    cnt = jnp.maximum(cnt_ref[0:1, :], 1.0)
    mean_t = sumt_ref[...] / cnt
    summary_t = jax.nn.sigmoid(mean_t)
    vt_ref[...] = jnp.dot(w_mat_ref[...], summary_t,
                          preferred_element_type=jnp.float32)
    w_ref[...] = jnp.broadcast_to(1.0 / cnt, (8, _BP))


def _loss_kernel(seg_ref, h_ref, vt_ref, w_ref, permf_ref, b_ref, out_ref):
    i = pl.program_id(0)

    @pl.when(i == 0)
    def _init():
        out_ref[...] = jnp.zeros_like(out_ref)

    seg = seg_ref[0, 0, :]
    a = jnp.dot(h_ref[...], vt_ref[...],
                preferred_element_type=jnp.float32)
    ids = jax.lax.broadcasted_iota(jnp.int32, (_TN, _BP), 1)
    oh1 = (ids == seg[:, None]).astype(jnp.float32)
    d_pos = jnp.sum(a * oh1, axis=1, keepdims=True)
    ps = jnp.sum(oh1 * permf_ref[0:1, :], axis=1, keepdims=True)
    oh2 = (ids.astype(jnp.float32) == ps).astype(jnp.float32)
    d_neg = jnp.sum(a * oh2, axis=1, keepdims=True)
    wn = jnp.sum(oh1 * w_ref[0:1, :], axis=1, keepdims=True)
    b0 = b_ref[0]
    dp = d_pos + b0
    dn = d_neg + b0

    def softplus(x):
        return jnp.maximum(x, 0.0) + jnp.log1p(jnp.exp(-jnp.abs(x)))

    out_ref[...] += jnp.sum(wn * (softplus(dp) - dp + softplus(dn)))


def kernel(h, W, b, segment_ids, perm):
    seg3 = segment_ids.astype(jnp.int32).reshape(_GRID, 1, _TN)

    seg_sum, cnt = pl.pallas_call(
        _segsum_kernel,
        grid=(_GRID,),
        in_specs=[
            pl.BlockSpec((1, 1, _TN), lambda i: (i, 0, 0)),
            pl.BlockSpec((_TN, _D), lambda i: (i, 0)),
        ],
        out_specs=[
            pl.BlockSpec((_BP, _D), lambda i: (0, 0)),
            pl.BlockSpec((8, _BP), lambda i: (0, 0)),
        ],
        out_shape=[
            jax.ShapeDtypeStruct((_BP, _D), jnp.float32),
            jax.ShapeDtypeStruct((8, _BP), jnp.float32),
        ],
    )(seg3, h)

    sum_t = seg_sum.T
    vt, w = pl.pallas_call(
        _table_kernel,
        in_specs=[
            pl.BlockSpec((_D, _BP), lambda: (0, 0)),
            pl.BlockSpec((8, _BP), lambda: (0, 0)),
            pl.BlockSpec((_D, _D), lambda: (0, 0)),
        ],
        out_specs=[
            pl.BlockSpec((_D, _BP), lambda: (0, 0)),
            pl.BlockSpec((8, _BP), lambda: (0, 0)),
        ],
        out_shape=[
            jax.ShapeDtypeStruct((_D, _BP), jnp.float32),
            jax.ShapeDtypeStruct((8, _BP), jnp.float32),
        ],
    )(sum_t, cnt, W)

    permf = jnp.concatenate(
        [perm.astype(jnp.float32), jnp.zeros((_BP - _B,), jnp.float32)])
    permf = jnp.broadcast_to(permf[None, :], (8, _BP))

    loss = pl.pallas_call(
        _loss_kernel,
        grid=(_GRID,),
        in_specs=[
            pl.BlockSpec((1, 1, _TN), lambda i: (i, 0, 0)),
            pl.BlockSpec((_TN, _D), lambda i: (i, 0)),
            pl.BlockSpec((_D, _BP), lambda i: (0, 0)),
            pl.BlockSpec((8, _BP), lambda i: (0, 0)),
            pl.BlockSpec((8, _BP), lambda i: (0, 0)),
            pl.BlockSpec(memory_space=pltpu.SMEM),
        ],
        out_specs=pl.BlockSpec((1, 1), lambda i: (0, 0)),
        out_shape=jax.ShapeDtypeStruct((1, 1), jnp.float32),
    )(seg3, h, vt, w, permf, b)

    return loss[0, 0]


# speedup vs baseline: 2.2591x; 1.1065x over previous
import jax
import jax.numpy as jnp
from jax.experimental import pallas as pl
from jax.experimental.pallas import tpu as pltpu
from jax.experimental.pallas import tpu_sc as plsc

_N = 100000
_D = 128
_B = 1000
_BP = 1024
_TN = 400
_GRID = _N // _TN

_NC = 2
_NS = 16
_NW = _NC * _NS
_NPAD = 102400
_CH = 128
_PER_W = _NPAD // _NW
_STEPS = _PER_W // _CH
_RPS = _BP // _NS


def _sc_segsum_body(h_hbm, seg_hbm, sum_out, cnt_out,
                    idx_v, rows_v, ones_v, zsum_v, zcnt_v, ssum, scnt):
    cid = jax.lax.axis_index("c")
    sid = jax.lax.axis_index("s")
    wid = sid * _NC + cid

    def fill_ones(i, carry):
        ones_v[i // 8, pl.ds((i % 8) * 16, 16)] = jnp.ones((16,), jnp.float32)
        return carry
    jax.lax.fori_loop(0, _CH * 8, fill_ones, 0)

    def zero_sum(i, carry):
        zsum_v[i // 8, pl.ds((i % 8) * 16, 16)] = jnp.zeros((16,), jnp.float32)
        return carry
    jax.lax.fori_loop(0, _RPS * 8, zero_sum, 0)

    def zero_cnt(i, carry):
        zcnt_v[i // 8, pl.ds((i % 8) * 16, 16)] = jnp.zeros((16,), jnp.float32)
        return carry
    jax.lax.fori_loop(0, _RPS * 8, zero_cnt, 0)

    r0 = sid * _RPS
    pltpu.sync_copy(zsum_v, ssum.at[pl.ds(r0, _RPS), :])
    pltpu.sync_copy(zcnt_v, scnt.at[pl.ds(r0, _RPS), :])
    plsc.subcore_barrier()

    base = wid * _PER_W

    def step(j, carry):
        off = base + j * _CH
        pltpu.sync_copy(seg_hbm.at[pl.ds(off, _CH)], idx_v)
        pltpu.sync_copy(h_hbm.at[pl.ds(off, _CH), :], rows_v)
        pltpu.sync_copy(rows_v, ssum.at[idx_v], add=True)
        pltpu.sync_copy(ones_v, scnt.at[idx_v], add=True)
        return carry
    jax.lax.fori_loop(0, _STEPS, step, 0)

    plsc.subcore_barrier()
    pltpu.sync_copy(ssum.at[pl.ds(r0, _RPS), :],
                    sum_out.at[cid, pl.ds(r0, _RPS), :])
    pltpu.sync_copy(scnt.at[pl.ds(r0, _RPS), :],
                    cnt_out.at[cid, pl.ds(r0, _RPS), :])


_sc_segsum = pl.kernel(
    _sc_segsum_body,
    out_type=[
        jax.ShapeDtypeStruct((_NC, _BP, _D), jnp.float32),
        jax.ShapeDtypeStruct((_NC, _BP, _D), jnp.float32),
    ],
    mesh=plsc.VectorSubcoreMesh(core_axis_name="c", subcore_axis_name="s",
                                num_cores=_NC, num_subcores=_NS),
    scratch_types=[
        pltpu.VMEM((_CH,), jnp.int32),
        pltpu.VMEM((_CH, _D), jnp.float32),
        pltpu.VMEM((_CH, _D), jnp.float32),
        pltpu.VMEM((_RPS, _D), jnp.float32),
        pltpu.VMEM((_RPS, _D), jnp.float32),
        pltpu.VMEM_SHARED((_BP, _D), jnp.float32),
        pltpu.VMEM_SHARED((_BP, _D), jnp.float32),
    ],
)


def _table_kernel(sumt_ref, cnt_ref, w_mat_ref, vt_ref, w_ref):
    cnt = jnp.maximum(cnt_ref[0:1, :] + cnt_ref[1:2, :], 1.0)
    mean_t = (sumt_ref[0] + sumt_ref[1]) / cnt
    summary_t = jax.nn.sigmoid(mean_t)
    vt_ref[...] = jnp.dot(w_mat_ref[...], summary_t,
                          preferred_element_type=jnp.float32)
    w_ref[...] = jnp.broadcast_to(1.0 / cnt, (8, _BP))


def _loss_kernel(seg_ref, h_ref, vt_ref, w_ref, permf_ref, b_ref, out_ref):
    i = pl.program_id(0)

    @pl.when(i == 0)
    def _init():
        out_ref[...] = jnp.zeros_like(out_ref)

    seg = seg_ref[0, 0, :]
    a = jnp.dot(h_ref[...], vt_ref[...],
                preferred_element_type=jnp.float32)
    ids = jax.lax.broadcasted_iota(jnp.int32, (_TN, _BP), 1)
    oh1 = (ids == seg[:, None]).astype(jnp.float32)
    d_pos = jnp.sum(a * oh1, axis=1, keepdims=True)
    ps = jnp.sum(oh1 * permf_ref[0:1, :], axis=1, keepdims=True)
    oh2 = (ids.astype(jnp.float32) == ps).astype(jnp.float32)
    d_neg = jnp.sum(a * oh2, axis=1, keepdims=True)
    wn = jnp.sum(oh1 * w_ref[0:1, :], axis=1, keepdims=True)
    b0 = b_ref[0]
    dp = d_pos + b0
    dn = d_neg + b0

    def softplus(x):
        return jnp.maximum(x, 0.0) + jnp.log1p(jnp.exp(-jnp.abs(x)))

    out_ref[...] += jnp.sum(wn * (softplus(dp) - dp + softplus(dn)))


def kernel(h, W, b, segment_ids, perm):
    seg32 = segment_ids.astype(jnp.int32)

    h_pad = jnp.concatenate(
        [h, jnp.zeros((_NPAD - _N, _D), h.dtype)], axis=0)
    seg_pad = jnp.concatenate(
        [seg32, jnp.full((_NPAD - _N,), _B, jnp.int32)])

    sums2, cnts2 = _sc_segsum(h_pad, seg_pad)

    sums2_t = jnp.transpose(sums2, (0, 2, 1))
    cnt8 = jnp.concatenate(
        [cnts2[:, :, 0], jnp.zeros((6, _BP), jnp.float32)], axis=0)

    vt, w = pl.pallas_call(
        _table_kernel,
        in_specs=[
            pl.BlockSpec((_NC, _D, _BP), lambda: (0, 0, 0)),
            pl.BlockSpec((8, _BP), lambda: (0, 0)),
            pl.BlockSpec((_D, _D), lambda: (0, 0)),
        ],
        out_specs=[
            pl.BlockSpec((_D, _BP), lambda: (0, 0)),
            pl.BlockSpec((8, _BP), lambda: (0, 0)),
        ],
        out_shape=[
            jax.ShapeDtypeStruct((_D, _BP), jnp.float32),
            jax.ShapeDtypeStruct((8, _BP), jnp.float32),
        ],
    )(sums2_t, cnt8, W)

    permf = jnp.concatenate(
        [perm.astype(jnp.float32), jnp.zeros((_BP - _B,), jnp.float32)])
    permf = jnp.broadcast_to(permf[None, :], (8, _BP))

    seg3 = seg32.reshape(_GRID, 1, _TN)
    loss = pl.pallas_call(
        _loss_kernel,
        grid=(_GRID,),
        in_specs=[
            pl.BlockSpec((1, 1, _TN), lambda i: (i, 0, 0)),
            pl.BlockSpec((_TN, _D), lambda i: (i, 0)),
            pl.BlockSpec((_D, _BP), lambda i: (0, 0)),
            pl.BlockSpec((8, _BP), lambda i: (0, 0)),
            pl.BlockSpec((8, _BP), lambda i: (0, 0)),
            pl.BlockSpec(memory_space=pltpu.SMEM),
        ],
        out_specs=pl.BlockSpec((1, 1), lambda i: (0, 0)),
        out_shape=jax.ShapeDtypeStruct((1, 1), jnp.float32),
    )(seg3, h, vt, w, permf, b)

    return loss[0, 0]
